# Initial kernel scaffold; baseline (speedup 1.0000x reference)
import jax
import jax.numpy as jnp
from jax.experimental import pallas as pl


def kernel(x, pos, edge_index, W1, b1, W2, b2, W3, b3, Wg, bg):
    raise NotImplementedError("write your pallas kernel here")


# speedup vs baseline: 1.1025x; 1.1025x over previous
import functools

import jax
import jax.numpy as jnp
from jax.experimental import pallas as pl
from jax.experimental.pallas import tpu as pltpu

_EB = 2048


def _edge_mlp_body(ga_ref, gu_ref, W2_ref, b2_ref, W3_ref, b3_ref, out_ref):
    h1 = jnp.maximum(ga_ref[...] - gu_ref[...], 0.0)
    h2 = jnp.maximum(
        jax.lax.dot_general(h1, W2_ref[...], (((1,), (0,)), ((), ())),
                            preferred_element_type=jnp.float32) + b2_ref[...],
        0.0)
    h3 = jnp.maximum(
        jax.lax.dot_general(h2, W3_ref[...], (((1,), (0,)), ((), ())),
                            preferred_element_type=jnp.float32) + b3_ref[...],
        0.0)
    out_ref[...] = h3


def _edge_mlp(ga, gu, W2, b2, W3, b3):
    E = ga.shape[0]
    grid = (E // _EB,)
    return pl.pallas_call(
        _edge_mlp_body,
        grid=grid,
        in_specs=[
            pl.BlockSpec((_EB, 64), lambda i: (i, 0)),
            pl.BlockSpec((_EB, 64), lambda i: (i, 0)),
            pl.BlockSpec((64, 128), lambda i: (0, 0)),
            pl.BlockSpec((1, 128), lambda i: (0, 0)),
            pl.BlockSpec((128, 128), lambda i: (0, 0)),
            pl.BlockSpec((1, 128), lambda i: (0, 0)),
        ],
        out_specs=pl.BlockSpec((_EB, 128), lambda i: (i, 0)),
        out_shape=jax.ShapeDtypeStruct((E, 128), jnp.float32),
    )(ga, gu, W2, b2, W3, b3)


def _global_mlp_body(agg_ref, Wg_ref, bg_ref, out_ref):
    o = jax.lax.dot_general(agg_ref[...], Wg_ref[...], (((1,), (0,)), ((), ())),
                            preferred_element_type=jnp.float32) + bg_ref[...]
    out_ref[...] = jnp.maximum(o, 0.0)


def _global_mlp(agg, Wg, bg):
    N = agg.shape[0]
    return pl.pallas_call(
        _global_mlp_body,
        grid=(N // 2000,),
        in_specs=[
            pl.BlockSpec((2000, 128), lambda i: (i, 0)),
            pl.BlockSpec((128, 128), lambda i: (0, 0)),
            pl.BlockSpec((1, 128), lambda i: (0, 0)),
        ],
        out_specs=pl.BlockSpec((2000, 128), lambda i: (i, 0)),
        out_shape=jax.ShapeDtypeStruct((N, 128), jnp.float32),
    )(agg, Wg, bg)


def kernel(x, pos, edge_index, W1, b1, W2, b2, W3, b3, Wg, bg):
    N = x.shape[0]
    E = edge_index.shape[1]
    Etot = E + N
    Epad = ((Etot + _EB - 1) // _EB) * _EB
    Npad = N + 16

    u = pos @ W1[128:]
    a = x @ W1[:128] + u + b1

    apad = jnp.zeros((Npad, 64), jnp.float32).at[:N].set(a)
    upad = jnp.zeros((Npad, 64), jnp.float32).at[:N].set(u)

    loops = jnp.arange(N, dtype=edge_index.dtype)
    pad = jnp.full((Epad - Etot,), N, dtype=edge_index.dtype)
    src = jnp.concatenate([edge_index[0], loops, pad])
    dst = jnp.concatenate([edge_index[1], loops, pad])

    ga = apad[src]
    gu = upad[dst]
    h3 = _edge_mlp(ga, gu, W2, b2.reshape(1, 128), W3, b3.reshape(1, 128))
    agg = jax.ops.segment_max(h3, dst, num_segments=Npad)
    agg = jnp.where(jnp.isfinite(agg), agg, 0.0)[:N]
    return _global_mlp(agg, Wg, bg.reshape(1, 128)).reshape(N, 128)


# speedup vs baseline: 1.9329x; 1.7532x over previous
import jax
import jax.numpy as jnp
from jax import lax
from jax.experimental import pallas as pl
from jax.experimental.pallas import tpu as pltpu
from jax.experimental.pallas import tpu_sc as plsc

N_NODES = 10000
NPAD = 10240
P_OWN = 320
NC, NS, NW = 2, 16, 32
EPAD = 332800
C_W = EPAD // NW
G = 104
NB_G = C_W // G
SCHUNK = 1024
NCHUNK = EPAD // SCHUNK
BATCH = 128
EB = 2560


def _mesh():
    return plsc.VectorSubcoreMesh(core_axis_name="c", subcore_axis_name="s",
                                  num_cores=NC, num_subcores=NS)


_SC_PARAMS = pltpu.CompilerParams(needs_layout_passes=False)


def _wid():
    return lax.axis_index("s") * NC + lax.axis_index("c")


def _sc_gather_body(a_hbm, u_hbm, src_hbm, dst_hbm, ga_hbm, gu_hbm,
                    sidx0, sidx1, didx0, didx1, arows0, arows1,
                    urows0, urows1, si0, si1, sg0, sg1, sw0, sw1):
    wid = _wid()
    cbase = wid * C_W
    sidx = (sidx0, sidx1)
    didx = (didx0, didx1)
    arows = (arows0, arows1)
    urows = (urows0, urows1)
    semi = (si0, si1)
    semg = (sg0, sg1)
    semw = (sw0, sw1)

    def idx_copies(k, s):
        base = cbase + k * G
        return (
            pltpu.make_async_copy(src_hbm.at[pl.ds(base, G)], sidx[s], semi[s]),
            pltpu.make_async_copy(dst_hbm.at[pl.ds(base, G)], didx[s], semi[s]),
        )

    def gather_copies(s):
        return (
            pltpu.make_async_copy(a_hbm.at[sidx[s]], arows[s], semg[s]),
            pltpu.make_async_copy(u_hbm.at[didx[s]], urows[s], semg[s]),
        )

    def wb_copies(k, s):
        base = cbase + k * G
        return (
            pltpu.make_async_copy(arows[s], ga_hbm.at[pl.ds(base, G)], semw[s]),
            pltpu.make_async_copy(urows[s], gu_hbm.at[pl.ds(base, G)], semw[s]),
        )

    def start(cs):
        for c in cs:
            c.start()

    def wait(cs):
        for c in cs:
            c.wait()

    start(idx_copies(0, 0))
    start(idx_copies(1, 1))
    wait(idx_copies(0, 0))
    start(gather_copies(0))

    def super_step(kk, _):
        for s in (0, 1):
            k = 2 * kk + s
            t = 1 - s

            @pl.when(k + 1 < NB_G)
            def _():
                wait(idx_copies(k + 1, t))

                @pl.when(k + 1 >= 2)
                def _():
                    wait(wb_copies(k - 1, t))

                start(gather_copies(t))

            wait(gather_copies(s))
            start(wb_copies(k, s))

            @pl.when(k + 2 < NB_G)
            def _():
                start(idx_copies(k + 2, s))

        return 0

    lax.fori_loop(0, NB_G // 2, super_step, 0)
    wait(wb_copies(NB_G - 2, 0))
    wait(wb_copies(NB_G - 1, 1))


def _sc_gather(apad, upad, src, dst):
    f = pl.kernel(
        _sc_gather_body,
        out_type=[jax.ShapeDtypeStruct((EPAD, 128), jnp.float32),
                  jax.ShapeDtypeStruct((EPAD, 128), jnp.float32)],
        mesh=_mesh(),
        scratch_types=[
            pltpu.VMEM((G,), jnp.int32), pltpu.VMEM((G,), jnp.int32),
            pltpu.VMEM((G,), jnp.int32), pltpu.VMEM((G,), jnp.int32),
            pltpu.VMEM((G, 128), jnp.float32), pltpu.VMEM((G, 128), jnp.float32),
            pltpu.VMEM((G, 128), jnp.float32), pltpu.VMEM((G, 128), jnp.float32),
            pltpu.SemaphoreType.DMA, pltpu.SemaphoreType.DMA,
            pltpu.SemaphoreType.DMA, pltpu.SemaphoreType.DMA,
            pltpu.SemaphoreType.DMA, pltpu.SemaphoreType.DMA,
        ],
        compiler_params=_SC_PARAMS,
    )
    return f(apad, upad, src, dst)


def _sc_segmax_body(h3_hbm, dst_hbm, agg_hbm,
                    dchunk, rows, acc, eid2, dloc3, sem0, sem1):
    wid = _wid()
    lo = wid * P_OWN
    hi = lo + P_OWN
    iota16 = lax.iota(jnp.int32, 16)
    zeros16 = jnp.zeros((16,), jnp.float32)
    zeros16i = jnp.zeros((16,), jnp.int32)

    def z_acc(i, _):
        for j in range(8):
            acc[i, pl.ds(j * 16, 16)] = zeros16
        return 0
    lax.fori_loop(0, P_OWN, z_acc, 0)

    for r in range(2):
        for j in range(8):
            eid2[r, pl.ds(j * 16, 16)] = zeros16i

    def process_batch(cnt):
        cp = pltpu.make_async_copy(h3_hbm.at[eid2.at[0]], rows, sem1)
        cp.start()
        cp.wait()

        def fold(e, _):
            q = e // 16
            l = e - q * 16
            dlv = dloc3[q, pl.ds(0, 16)]
            dl = jnp.min(jnp.where(iota16 == l, dlv,
                                   jnp.full((16,), P_OWN, jnp.int32)))
            for j in range(8):
                sl = pl.ds(j * 16, 16)
                acc[dl, sl] = jnp.maximum(acc[dl, sl], rows[e, sl])
            return 0
        lax.fori_loop(0, cnt, fold, 0)

    def scan_chunk(c, fill):
        cbase = c * SCHUNK
        cp = pltpu.make_async_copy(dst_hbm.at[pl.ds(cbase, SCHUNK)], dchunk,
                                   sem0)
        cp.start()
        cp.wait()

        def group(g, fill):
            d = dchunk[pl.ds(g * 16, 16)]
            m = (d >= lo) & (d < hi)

            def matched(fl):
                mi = jnp.where(m, 1, 0).astype(jnp.int32)
                pos = plsc.cumsum(mi) - mi + fl
                eids = jnp.full((16,), cbase + g * 16, jnp.int32) + iota16
                plsc.store_scatter(
                    eid2,
                    [lax.shift_right_logical(pos, 7), pos & 127],
                    eids, mask=m)
                plsc.store_scatter(
                    dloc3,
                    [lax.shift_right_logical(pos, 4), pos & 15],
                    d - lo, mask=m)
                return fl + jnp.sum(mi)

            fill = lax.cond(jnp.any(m), matched, lambda fl: fl, fill)

            def drain(fl):
                process_batch(jnp.int32(BATCH))
                eid2[0, pl.ds(0, 16)] = eid2[1, pl.ds(0, 16)]
                dloc3[0, pl.ds(0, 16)] = dloc3[8, pl.ds(0, 16)]
                return fl - BATCH

            return lax.cond(fill >= BATCH, drain, lambda fl: fl, fill)

        return lax.fori_loop(0, SCHUNK // 16, group, fill)

    fill = lax.fori_loop(0, NCHUNK, scan_chunk, jnp.int32(0))

    @pl.when(fill > 0)
    def _():
        process_batch(fill)

    cp = pltpu.make_async_copy(acc, agg_hbm.at[pl.ds(wid * P_OWN, P_OWN)],
                               sem0)
    cp.start()
    cp.wait()


def _sc_segmax(h3, dst):
    f = pl.kernel(
        _sc_segmax_body,
        out_type=jax.ShapeDtypeStruct((NPAD, 128), jnp.float32),
        mesh=_mesh(),
        scratch_types=[
            pltpu.VMEM((SCHUNK,), jnp.int32),
            pltpu.VMEM((BATCH, 128), jnp.float32),
            pltpu.VMEM((P_OWN, 128), jnp.float32),
            pltpu.VMEM((2, 128), jnp.int32),
            pltpu.VMEM((16, 16), jnp.int32),
            pltpu.SemaphoreType.DMA, pltpu.SemaphoreType.DMA,
        ],
        compiler_params=_SC_PARAMS,
    )
    return f(h3, dst)


def _node_pre_body(x_ref, pos_ref, W1a_ref, W1b_ref, b1_ref, a_ref, u_ref):
    u = jax.lax.dot_general(pos_ref[...], W1b_ref[...],
                            (((1,), (0,)), ((), ())),
                            preferred_element_type=jnp.float32)
    a = jax.lax.dot_general(x_ref[...], W1a_ref[...], (((1,), (0,)), ((), ())),
                            preferred_element_type=jnp.float32)
    u_ref[...] = u
    a_ref[...] = a + u + b1_ref[...]


def _node_pre(x, pos, W1a, W1b, b1):
    n = x.shape[0]
    return pl.pallas_call(
        _node_pre_body,
        grid=(n // 2000,),
        in_specs=[
            pl.BlockSpec((2000, 128), lambda i: (i, 0)),
            pl.BlockSpec((2000, 3), lambda i: (i, 0)),
            pl.BlockSpec((128, 64), lambda i: (0, 0)),
            pl.BlockSpec((3, 64), lambda i: (0, 0)),
            pl.BlockSpec((1, 64), lambda i: (0, 0)),
        ],
        out_specs=[pl.BlockSpec((2000, 64), lambda i: (i, 0)),
                   pl.BlockSpec((2000, 64), lambda i: (i, 0))],
        out_shape=[jax.ShapeDtypeStruct((n, 64), jnp.float32),
                   jax.ShapeDtypeStruct((n, 64), jnp.float32)],
    )(x, pos, W1a, W1b, b1)


def _edge_mlp_body(ga_ref, gu_ref, W2_ref, b2_ref, W3_ref, b3_ref, out_ref):
    h1 = jnp.maximum(ga_ref[...] - gu_ref[...], 0.0)
    h2 = jnp.maximum(
        jax.lax.dot_general(h1, W2_ref[...], (((1,), (0,)), ((), ())),
                            preferred_element_type=jnp.float32) + b2_ref[...],
        0.0)
    h3 = jnp.maximum(
        jax.lax.dot_general(h2, W3_ref[...], (((1,), (0,)), ((), ())),
                            preferred_element_type=jnp.float32) + b3_ref[...],
        0.0)
    out_ref[...] = h3


def _edge_mlp(ga, gu, W2, b2, W3, b3):
    E = ga.shape[0]
    return pl.pallas_call(
        _edge_mlp_body,
        grid=(E // EB,),
        in_specs=[
            pl.BlockSpec((EB, 128), lambda i: (i, 0)),
            pl.BlockSpec((EB, 128), lambda i: (i, 0)),
            pl.BlockSpec((128, 128), lambda i: (0, 0)),
            pl.BlockSpec((1, 128), lambda i: (0, 0)),
            pl.BlockSpec((128, 128), lambda i: (0, 0)),
            pl.BlockSpec((1, 128), lambda i: (0, 0)),
        ],
        out_specs=pl.BlockSpec((EB, 128), lambda i: (i, 0)),
        out_shape=jax.ShapeDtypeStruct((E, 128), jnp.float32),
    )(ga, gu, W2, b2, W3, b3)


def _global_mlp_body(agg_ref, Wg_ref, bg_ref, out_ref):
    o = jax.lax.dot_general(agg_ref[...], Wg_ref[...], (((1,), (0,)), ((), ())),
                            preferred_element_type=jnp.float32) + bg_ref[...]
    out_ref[...] = jnp.maximum(o, 0.0)


def _global_mlp(agg, Wg, bg):
    n = agg.shape[0]
    return pl.pallas_call(
        _global_mlp_body,
        grid=(n // 2000,),
        in_specs=[
            pl.BlockSpec((2000, 128), lambda i: (i, 0)),
            pl.BlockSpec((128, 128), lambda i: (0, 0)),
            pl.BlockSpec((1, 128), lambda i: (0, 0)),
        ],
        out_specs=pl.BlockSpec((2000, 128), lambda i: (i, 0)),
        out_shape=jax.ShapeDtypeStruct((n, 128), jnp.float32),
    )(agg, Wg, bg)


def kernel(x, pos, edge_index, W1, b1, W2, b2, W3, b3, Wg, bg):
    N = x.shape[0]
    E = edge_index.shape[1]
    Etot = E + N

    a, u = _node_pre(x, pos, W1[:128], W1[128:], b1.reshape(1, 64))

    apad = jnp.zeros((NPAD, 128), jnp.float32).at[:N, :64].set(a)
    upad = jnp.zeros((NPAD, 128), jnp.float32).at[:N, :64].set(u)
    W2e = jnp.zeros((128, 128), jnp.float32).at[:64].set(W2)

    loops = jnp.arange(N, dtype=jnp.int32)
    npads = EPAD - Etot
    pad_src = jnp.arange(npads, dtype=jnp.int32) % N
    pad_dst = N + (jnp.arange(npads, dtype=jnp.int32) % (NPAD - N))
    ei = edge_index.astype(jnp.int32)
    src = jnp.concatenate([ei[0], loops, pad_src])
    dst = jnp.concatenate([ei[1], loops, pad_dst])

    ga, gu = _sc_gather(apad, upad, src, dst)
    h3 = _edge_mlp(ga, gu, W2e, b2.reshape(1, 128), W3, b3.reshape(1, 128))
    agg = _sc_segmax(h3, dst)[:N]
    return _global_mlp(agg, Wg, bg.reshape(1, 128)).reshape(N, 128)


# speedup vs baseline: 2.0029x; 1.0362x over previous
import jax
import jax.numpy as jnp
from jax import lax
from jax.experimental import pallas as pl
from jax.experimental.pallas import tpu as pltpu
from jax.experimental.pallas import tpu_sc as plsc

N_NODES = 10000
NPAD = 10240
P_OWN = 320
NC, NS, NW = 2, 16, 32
EPAD = 332800
C_W = EPAD // NW
G = 104
NB_G = C_W // G
SCHUNK = 1024
NCHUNK = EPAD // SCHUNK
BATCH = 128
EB = 2560


def _mesh():
    return plsc.VectorSubcoreMesh(core_axis_name="c", subcore_axis_name="s",
                                  num_cores=NC, num_subcores=NS)


_SC_PARAMS = pltpu.CompilerParams(needs_layout_passes=False)


def _wid():
    return lax.axis_index("s") * NC + lax.axis_index("c")


def _sc_gather_body(a_hbm, u_hbm, src_hbm, dst_hbm, ga_hbm, gu_hbm,
                    sidx0, sidx1, didx0, didx1, arows0, arows1,
                    urows0, urows1, si0, si1, sg0, sg1, sw0, sw1):
    wid = _wid()
    cbase = wid * C_W
    sidx = (sidx0, sidx1)
    didx = (didx0, didx1)
    arows = (arows0, arows1)
    urows = (urows0, urows1)
    semi = (si0, si1)
    semg = (sg0, sg1)
    semw = (sw0, sw1)

    def idx_copies(k, s):
        base = cbase + k * G
        return (
            pltpu.make_async_copy(src_hbm.at[pl.ds(base, G)], sidx[s], semi[s]),
            pltpu.make_async_copy(dst_hbm.at[pl.ds(base, G)], didx[s], semi[s]),
        )

    def gather_copies(s):
        return (
            pltpu.make_async_copy(a_hbm.at[sidx[s]], arows[s], semg[s]),
            pltpu.make_async_copy(u_hbm.at[didx[s]], urows[s], semg[s]),
        )

    def wb_copies(k, s):
        base = cbase + k * G
        return (
            pltpu.make_async_copy(arows[s], ga_hbm.at[pl.ds(base, G)], semw[s]),
            pltpu.make_async_copy(urows[s], gu_hbm.at[pl.ds(base, G)], semw[s]),
        )

    def start(cs):
        for c in cs:
            c.start()

    def wait(cs):
        for c in cs:
            c.wait()

    start(idx_copies(0, 0))
    start(idx_copies(1, 1))
    wait(idx_copies(0, 0))
    start(gather_copies(0))

    def super_step(kk, _):
        for s in (0, 1):
            k = 2 * kk + s
            t = 1 - s

            @pl.when(k + 1 < NB_G)
            def _():
                wait(idx_copies(k + 1, t))

                @pl.when(k + 1 >= 2)
                def _():
                    wait(wb_copies(k - 1, t))

                start(gather_copies(t))

            wait(gather_copies(s))
            start(wb_copies(k, s))

            @pl.when(k + 2 < NB_G)
            def _():
                start(idx_copies(k + 2, s))

        return 0

    lax.fori_loop(0, NB_G // 2, super_step, 0)
    wait(wb_copies(NB_G - 2, 0))
    wait(wb_copies(NB_G - 1, 1))


def _sc_gather(apad, upad, src, dst):
    f = pl.kernel(
        _sc_gather_body,
        out_type=[jax.ShapeDtypeStruct((EPAD, 128), jnp.float32),
                  jax.ShapeDtypeStruct((EPAD, 128), jnp.float32)],
        mesh=_mesh(),
        scratch_types=[
            pltpu.VMEM((G,), jnp.int32), pltpu.VMEM((G,), jnp.int32),
            pltpu.VMEM((G,), jnp.int32), pltpu.VMEM((G,), jnp.int32),
            pltpu.VMEM((G, 128), jnp.float32), pltpu.VMEM((G, 128), jnp.float32),
            pltpu.VMEM((G, 128), jnp.float32), pltpu.VMEM((G, 128), jnp.float32),
            pltpu.SemaphoreType.DMA, pltpu.SemaphoreType.DMA,
            pltpu.SemaphoreType.DMA, pltpu.SemaphoreType.DMA,
            pltpu.SemaphoreType.DMA, pltpu.SemaphoreType.DMA,
        ],
        compiler_params=_SC_PARAMS,
    )
    return f(apad, upad, src, dst)


def _sc_segmax_body(h3_hbm, dst_hbm, gm_hbm, agg_hbm,
                    dchunk, gmrows, rows, acc, eid2, dloc3, sem0, sem1):
    wid = _wid()
    lo = wid * P_OWN
    hi = lo + P_OWN
    iota16 = lax.iota(jnp.int32, 16)
    zeros16 = jnp.zeros((16,), jnp.float32)
    zeros16i = jnp.zeros((16,), jnp.int32)

    def z_acc(i, _):
        for j in range(8):
            acc[i, pl.ds(j * 16, 16)] = zeros16
        return 0
    lax.fori_loop(0, P_OWN, z_acc, 0)

    for r in range(2):
        for j in range(8):
            eid2[r, pl.ds(j * 16, 16)] = zeros16i

    def process_batch(cnt):
        cp = pltpu.make_async_copy(h3_hbm.at[eid2.at[0]], rows, sem1)
        cp.start()
        cp.wait()

        def foldq(q, _):
            dlv = dloc3[q, pl.ds(0, 16)]
            base = q * 16
            for l in range(16):
                e = base + l

                @pl.when(e < cnt)
                def _():
                    dl = dlv[l]
                    for j in range(8):
                        sl = pl.ds(j * 16, 16)
                        acc[dl, sl] = jnp.maximum(acc[dl, sl], rows[e, sl])
            return 0
        lax.fori_loop(0, (cnt + 15) >> 4, foldq, 0)

    def scan_chunk(c, fill):
        cbase = c * SCHUNK
        cp = pltpu.make_async_copy(dst_hbm.at[pl.ds(cbase, SCHUNK)], dchunk,
                                   sem0)
        cp.start()
        cp2 = pltpu.make_async_copy(
            gm_hbm.at[pl.ds(c * (SCHUNK // 16), SCHUNK // 16)], gmrows, sem1)
        cp2.start()
        cp.wait()
        cp2.wait()

        def group(g, fill):
            gmv = gmrows[g, pl.ds(0, 16)]
            hit = lax.shift_right_logical(gmv[0], wid) & 1

            def matched(fl):
                d = dchunk[pl.ds(g * 16, 16)]
                m = (d >= lo) & (d < hi)
                mi = jnp.where(m, 1, 0).astype(jnp.int32)
                pos = plsc.cumsum(mi) - mi + fl
                eids = jnp.full((16,), cbase + g * 16, jnp.int32) + iota16
                plsc.store_scatter(
                    eid2,
                    [lax.shift_right_logical(pos, 7), pos & 127],
                    eids, mask=m)
                plsc.store_scatter(
                    dloc3,
                    [lax.shift_right_logical(pos, 4), pos & 15],
                    d - lo, mask=m)
                return fl + jnp.sum(mi)

            fill = lax.cond(hit == 1, matched, lambda fl: fl, fill)

            def drain(fl):
                process_batch(jnp.int32(BATCH))
                eid2[0, pl.ds(0, 16)] = eid2[1, pl.ds(0, 16)]
                dloc3[0, pl.ds(0, 16)] = dloc3[8, pl.ds(0, 16)]
                return fl - BATCH

            return lax.cond(fill >= BATCH, drain, lambda fl: fl, fill)

        return lax.fori_loop(0, SCHUNK // 16, group, fill)

    fill = lax.fori_loop(0, NCHUNK, scan_chunk, jnp.int32(0))

    @pl.when(fill > 0)
    def _():
        process_batch(fill)

    cp = pltpu.make_async_copy(acc, agg_hbm.at[pl.ds(wid * P_OWN, P_OWN)],
                               sem0)
    cp.start()
    cp.wait()


def _sc_segmax(h3, dst, gm):
    f = pl.kernel(
        _sc_segmax_body,
        out_type=jax.ShapeDtypeStruct((NPAD, 128), jnp.float32),
        mesh=_mesh(),
        scratch_types=[
            pltpu.VMEM((SCHUNK,), jnp.int32),
            pltpu.VMEM((SCHUNK // 16, 16), jnp.int32),
            pltpu.VMEM((BATCH, 128), jnp.float32),
            pltpu.VMEM((P_OWN, 128), jnp.float32),
            pltpu.VMEM((2, 128), jnp.int32),
            pltpu.VMEM((16, 16), jnp.int32),
            pltpu.SemaphoreType.DMA, pltpu.SemaphoreType.DMA,
        ],
        compiler_params=_SC_PARAMS,
    )
    return f(h3, dst, gm)


def _group_mask_body(d_ref, gm_ref):
    d = d_ref[...]
    owner = lax.shift_right_logical(
        (lax.shift_right_logical(d, 6) * 52429), 18)
    bits = lax.shift_left(jnp.ones_like(owner), owner)
    w = bits[:, :8] | bits[:, 8:]
    w = w[:, :4] | w[:, 4:]
    w = w[:, :2] | w[:, 2:]
    w = w[:, :1] | w[:, 1:]
    gm_ref[...] = jnp.broadcast_to(w, d.shape)


def _group_mask(d2):
    ng = d2.shape[0]
    return pl.pallas_call(
        _group_mask_body,
        grid=(ng // 2080,),
        in_specs=[pl.BlockSpec((2080, 16), lambda i: (i, 0))],
        out_specs=pl.BlockSpec((2080, 16), lambda i: (i, 0)),
        out_shape=jax.ShapeDtypeStruct((ng, 16), jnp.int32),
    )(d2)


def _node_pre_body(x_ref, pos_ref, W1a_ref, W1b_ref, b1_ref, a_ref, u_ref):
    u = jax.lax.dot_general(pos_ref[...], W1b_ref[...],
                            (((1,), (0,)), ((), ())),
                            preferred_element_type=jnp.float32)
    a = jax.lax.dot_general(x_ref[...], W1a_ref[...], (((1,), (0,)), ((), ())),
                            preferred_element_type=jnp.float32)
    u_ref[...] = u
    a_ref[...] = a + u + b1_ref[...]


def _node_pre(x, pos, W1a, W1b, b1):
    n = x.shape[0]
    return pl.pallas_call(
        _node_pre_body,
        grid=(n // 2000,),
        in_specs=[
            pl.BlockSpec((2000, 128), lambda i: (i, 0)),
            pl.BlockSpec((2000, 3), lambda i: (i, 0)),
            pl.BlockSpec((128, 64), lambda i: (0, 0)),
            pl.BlockSpec((3, 64), lambda i: (0, 0)),
            pl.BlockSpec((1, 64), lambda i: (0, 0)),
        ],
        out_specs=[pl.BlockSpec((2000, 64), lambda i: (i, 0)),
                   pl.BlockSpec((2000, 64), lambda i: (i, 0))],
        out_shape=[jax.ShapeDtypeStruct((n, 64), jnp.float32),
                   jax.ShapeDtypeStruct((n, 64), jnp.float32)],
    )(x, pos, W1a, W1b, b1)


def _edge_mlp_body(ga_ref, gu_ref, W2_ref, b2_ref, W3_ref, b3_ref, out_ref):
    h1 = jnp.maximum(ga_ref[...] - gu_ref[...], 0.0)
    h2 = jnp.maximum(
        jax.lax.dot_general(h1, W2_ref[...], (((1,), (0,)), ((), ())),
                            preferred_element_type=jnp.float32) + b2_ref[...],
        0.0)
    h3 = jnp.maximum(
        jax.lax.dot_general(h2, W3_ref[...], (((1,), (0,)), ((), ())),
                            preferred_element_type=jnp.float32) + b3_ref[...],
        0.0)
    out_ref[...] = h3


def _edge_mlp(ga, gu, W2, b2, W3, b3):
    E = ga.shape[0]
    return pl.pallas_call(
        _edge_mlp_body,
        grid=(E // EB,),
        in_specs=[
            pl.BlockSpec((EB, 128), lambda i: (i, 0)),
            pl.BlockSpec((EB, 128), lambda i: (i, 0)),
            pl.BlockSpec((128, 128), lambda i: (0, 0)),
            pl.BlockSpec((1, 128), lambda i: (0, 0)),
            pl.BlockSpec((128, 128), lambda i: (0, 0)),
            pl.BlockSpec((1, 128), lambda i: (0, 0)),
        ],
        out_specs=pl.BlockSpec((EB, 128), lambda i: (i, 0)),
        out_shape=jax.ShapeDtypeStruct((E, 128), jnp.float32),
    )(ga, gu, W2, b2, W3, b3)


def _global_mlp_body(agg_ref, Wg_ref, bg_ref, out_ref):
    o = jax.lax.dot_general(agg_ref[...], Wg_ref[...], (((1,), (0,)), ((), ())),
                            preferred_element_type=jnp.float32) + bg_ref[...]
    out_ref[...] = jnp.maximum(o, 0.0)


def _global_mlp(agg, Wg, bg):
    n = agg.shape[0]
    return pl.pallas_call(
        _global_mlp_body,
        grid=(n // 2000,),
        in_specs=[
            pl.BlockSpec((2000, 128), lambda i: (i, 0)),
            pl.BlockSpec((128, 128), lambda i: (0, 0)),
            pl.BlockSpec((1, 128), lambda i: (0, 0)),
        ],
        out_specs=pl.BlockSpec((2000, 128), lambda i: (i, 0)),
        out_shape=jax.ShapeDtypeStruct((n, 128), jnp.float32),
    )(agg, Wg, bg)


def kernel(x, pos, edge_index, W1, b1, W2, b2, W3, b3, Wg, bg):
    N = x.shape[0]
    E = edge_index.shape[1]
    Etot = E + N

    a, u = _node_pre(x, pos, W1[:128], W1[128:], b1.reshape(1, 64))

    apad = jnp.zeros((NPAD, 128), jnp.float32).at[:N, :64].set(a)
    upad = jnp.zeros((NPAD, 128), jnp.float32).at[:N, :64].set(u)
    W2e = jnp.zeros((128, 128), jnp.float32).at[:64].set(W2)

    loops = jnp.arange(N, dtype=jnp.int32)
    npads = EPAD - Etot
    pad_src = jnp.arange(npads, dtype=jnp.int32) % N
    pad_dst = N + (jnp.arange(npads, dtype=jnp.int32) % (NPAD - N))
    ei = edge_index.astype(jnp.int32)
    src = jnp.concatenate([ei[0], loops, pad_src])
    dst = jnp.concatenate([ei[1], loops, pad_dst])

    ga, gu = _sc_gather(apad, upad, src, dst)
    h3 = _edge_mlp(ga, gu, W2e, b2.reshape(1, 128), W3, b3.reshape(1, 128))
    gm = _group_mask(dst.reshape(EPAD // 16, 16))
    agg = _sc_segmax(h3, dst, gm)[:N]
    return _global_mlp(agg, Wg, bg.reshape(1, 128)).reshape(N, 128)
